# Initial kernel scaffold; baseline (speedup 1.0000x reference)
import jax
import jax.numpy as jnp
from jax.experimental import pallas as pl


def kernel(x, action, edge_index, batch, W1, b1, W2, b2, Wout, bout):
    raise NotImplementedError("write your pallas kernel here")


# speedup vs baseline: 4.5730x; 4.5730x over previous
import functools

import jax
import jax.numpy as jnp
from jax import lax
from jax.experimental import pallas as pl
from jax.experimental.pallas import tpu as pltpu
from jax.experimental.pallas import tpu_sc as plsc

N = 10000
E = 160000
G = 8
NC = 2
NS = 16
EB = 128
DC = 128
C1 = 4
C2 = 8
NPAD = N + 16
RPT = 624
RTAIL = N - NS * RPT

S_AGG = -(-E // (NS * EB))
E_AGG = NS * S_AGG * EB
S_DEG = -(-E // (NC * NS * EB))
E_DEG = NC * NS * S_DEG * EB

RB = 1000
NB = N // RB

_MESH = plsc.VectorSubcoreMesh(core_axis_name="c", subcore_axis_name="s",
                               num_cores=NC, num_subcores=NS)


@functools.partial(
    pl.kernel,
    out_type=jax.ShapeDtypeStruct((NC * N, DC), jnp.float32),
    mesh=_MESH,
    scratch_types=[
        pltpu.VMEM_SHARED((NPAD, DC), jnp.float32),
        pltpu.VMEM((S_DEG, EB), jnp.int32),
        pltpu.VMEM((EB, DC), jnp.float32),
    ],
)
def _deg_kernel(dst_hbm, zeros_hbm, ones_hbm, out_hbm, acc, dst_v, ones_v):
    cid = lax.axis_index("c")
    sid = lax.axis_index("s")
    wid = cid * NS + sid
    pltpu.sync_copy(zeros_hbm.at[pl.ds(sid * RPT, RPT), :],
                    acc.at[pl.ds(sid * RPT, RPT), :])

    @pl.when(sid == 0)
    def _():
        tail = NPAD - NS * RPT
        pltpu.sync_copy(zeros_hbm.at[pl.ds(NS * RPT, tail), :],
                        acc.at[pl.ds(NS * RPT, tail), :])

    pltpu.sync_copy(ones_hbm, ones_v)
    pltpu.sync_copy(dst_hbm.at[wid], dst_v)
    plsc.subcore_barrier()

    def step(j, carry):
        pltpu.sync_copy(ones_v, acc.at[dst_v.at[j]], add=True)
        return carry

    lax.fori_loop(0, S_DEG, step, 0)
    plsc.subcore_barrier()
    pltpu.sync_copy(acc.at[pl.ds(sid * RPT, RPT), :],
                    out_hbm.at[pl.ds(cid * N + sid * RPT, RPT), :])

    @pl.when(sid == 0)
    def _():
        pltpu.sync_copy(acc.at[pl.ds(NS * RPT, RTAIL), :],
                        out_hbm.at[pl.ds(cid * N + NS * RPT, RTAIL), :])


def _make_agg(c_total):
    c_per_sc = c_total // NC

    @functools.partial(
        pl.kernel,
        out_type=jax.ShapeDtypeStruct((c_total * N, DC), jnp.float32),
        mesh=_MESH,
        scratch_types=[
            pltpu.VMEM_SHARED((NPAD, DC), jnp.float32),
            pltpu.VMEM((S_AGG, EB), jnp.int32),
            pltpu.VMEM((S_AGG, EB), jnp.int32),
            pltpu.VMEM((EB, DC), jnp.float32),
            pltpu.SemaphoreType.DMA,
        ],
    )
    def agg(table_hbm, srcg_hbm, dst_hbm, out_hbm, acc, src_v, dst_v, rows_v,
            sem):
        cid = lax.axis_index("c")
        sid = lax.axis_index("s")
        pltpu.sync_copy(dst_hbm.at[sid], dst_v)
        for kc in range(c_per_sc):
            c = cid * c_per_sc + kc
            base = c * N
            pltpu.sync_copy(table_hbm.at[pl.ds(base + sid * RPT, RPT), :],
                            acc.at[pl.ds(sid * RPT, RPT), :])

            @pl.when(sid == 0)
            def _():
                pltpu.sync_copy(table_hbm.at[pl.ds(base + NS * RPT, RTAIL), :],
                                acc.at[pl.ds(NS * RPT, RTAIL), :])

            pltpu.sync_copy(srcg_hbm.at[c * NS + sid], src_v)
            plsc.subcore_barrier()

            def step(j, carry):
                pltpu.async_copy(table_hbm.at[src_v.at[j]], rows_v,
                                 sem).wait()
                pltpu.sync_copy(rows_v, acc.at[dst_v.at[j]], add=True)
                return carry

            lax.fori_loop(0, S_AGG, step, 0)
            plsc.subcore_barrier()
            pltpu.sync_copy(acc.at[pl.ds(sid * RPT, RPT), :],
                            out_hbm.at[pl.ds(base + sid * RPT, RPT), :])

            @pl.when(sid == 0)
            def _():
                pltpu.sync_copy(acc.at[pl.ds(NS * RPT, RTAIL), :],
                                out_hbm.at[pl.ds(base + NS * RPT, RTAIL), :])

            plsc.subcore_barrier()

    return agg


_agg1 = _make_agg(C1)
_agg2 = _make_agg(C2)


def _leaky(v):
    return jnp.where(v >= 0, v, 0.01 * v)


def _prep_body(x_ref, a_ref, degp_ref, xp_ref, dinv_ref):
    deg = 1.0 + degp_ref[0, :, 0:1] + degp_ref[1, :, 0:1]
    dinv = lax.rsqrt(deg)
    xin = jnp.concatenate([x_ref[...], a_ref[...]], axis=1) * dinv
    for c in range(C1):
        xp_ref[c] = xin[:, c * DC:(c + 1) * DC]
    dinv_ref[...] = jnp.broadcast_to(dinv, (RB, 16))


def _mid_body(z1_ref, dinv_ref, w1_ref, b1_ref, w2_ref, out_ref):
    dinv = dinv_ref[:, 0:1]
    u1 = jnp.concatenate([z1_ref[c] for c in range(C1)], axis=1) * dinv
    h1 = _leaky(jnp.dot(u1, w1_ref[...],
                        preferred_element_type=jnp.float32) + b1_ref[...])
    m2 = jnp.dot(h1, w2_ref[...], preferred_element_type=jnp.float32) * dinv
    for c in range(C2):
        out_ref[c] = m2[:, c * DC:(c + 1) * DC]


def _fin_body(z2_ref, dinv_ref, b2_ref, wo_ref, bo_ref, batch_ref, out_ref,
              sums_sc, cnts_sc):
    i = pl.program_id(0)

    @pl.when(i == 0)
    def _():
        sums_sc[...] = jnp.zeros_like(sums_sc)
        cnts_sc[...] = jnp.zeros_like(cnts_sc)

    dinv = dinv_ref[:, 0:1]
    u2 = jnp.concatenate([z2_ref[c] for c in range(C2)], axis=1) * dinv
    h2 = _leaky(u2 + b2_ref[...])
    h3 = _leaky(jnp.dot(h2, wo_ref[...],
                        preferred_element_type=jnp.float32) + bo_ref[...])
    b = batch_ref[0, 0, :]
    oh = (b[:, None] == jax.lax.broadcasted_iota(jnp.int32, (1, G), 1)
          ).astype(jnp.float32)
    sums_sc[...] += lax.dot_general(oh, h3, (((0,), (0,)), ((), ())),
                                    preferred_element_type=jnp.float32)
    cnts_sc[...] += jnp.broadcast_to(jnp.sum(oh, axis=0)[:, None], (G, 512))

    @pl.when(i == NB - 1)
    def _():
        out_ref[...] = sums_sc[...] / jnp.maximum(cnts_sc[...], 1.0)


def kernel(x, action, edge_index, batch, W1, b1, W2, b2, Wout, bout):
    f32 = jnp.float32
    src = edge_index[0].astype(jnp.int32)
    dst = edge_index[1].astype(jnp.int32)

    src_p = jnp.concatenate([src, jnp.zeros((E_AGG - E,), jnp.int32)])
    dst_p = jnp.concatenate([dst, jnp.full((E_DEG - E,), N, jnp.int32)])
    src_r = src_p.reshape(NS, S_AGG, EB)
    srcg = (src_r[None] + (jnp.arange(C2, dtype=jnp.int32) * N)[:, None, None,
                                                                None])
    srcg = srcg.reshape(C2 * NS, S_AGG, EB)
    dst_agg = dst_p[:E_AGG].reshape(NS, S_AGG, EB)
    dst_deg = dst_p.reshape(NC * NS, S_DEG, EB)

    degp = _deg_kernel(dst_deg, jnp.zeros((NPAD, DC), f32),
                       jnp.ones((EB, DC), f32))
    degp = degp.reshape(NC, N, DC)

    xp, dinv16 = pl.pallas_call(
        _prep_body,
        grid=(NB,),
        in_specs=[
            pl.BlockSpec((RB, 256), lambda i: (i, 0)),
            pl.BlockSpec((RB, 256), lambda i: (i, 0)),
            pl.BlockSpec((NC, RB, DC), lambda i: (0, i, 0)),
        ],
        out_specs=[
            pl.BlockSpec((C1, RB, DC), lambda i: (0, i, 0)),
            pl.BlockSpec((RB, 16), lambda i: (i, 0)),
        ],
        out_shape=[
            jax.ShapeDtypeStruct((C1, N, DC), f32),
            jax.ShapeDtypeStruct((N, 16), f32),
        ],
    )(x, action, degp)

    z1 = _agg1(xp.reshape(C1 * N, DC), srcg[:C1 * NS], dst_agg)

    h2p = pl.pallas_call(
        _mid_body,
        grid=(NB,),
        in_specs=[
            pl.BlockSpec((C1, RB, DC), lambda i: (0, i, 0)),
            pl.BlockSpec((RB, 16), lambda i: (i, 0)),
            pl.BlockSpec((512, 1024), lambda i: (0, 0)),
            pl.BlockSpec((1, 1024), lambda i: (0, 0)),
            pl.BlockSpec((1024, 1024), lambda i: (0, 0)),
        ],
        out_specs=pl.BlockSpec((C2, RB, DC), lambda i: (0, i, 0)),
        out_shape=jax.ShapeDtypeStruct((C2, N, DC), f32),
    )(z1.reshape(C1, N, DC), dinv16, W1, b1.reshape(1, 1024), W2)

    z2 = _agg2(h2p.reshape(C2 * N, DC), srcg, dst_agg)

    out = pl.pallas_call(
        _fin_body,
        grid=(NB,),
        in_specs=[
            pl.BlockSpec((C2, RB, DC), lambda i: (0, i, 0)),
            pl.BlockSpec((RB, 16), lambda i: (i, 0)),
            pl.BlockSpec((1, 1024), lambda i: (0, 0)),
            pl.BlockSpec((1024, 512), lambda i: (0, 0)),
            pl.BlockSpec((1, 512), lambda i: (0, 0)),
            pl.BlockSpec((1, 1, RB), lambda i: (i, 0, 0)),
        ],
        out_specs=pl.BlockSpec((G, 512), lambda i: (0, 0)),
        out_shape=jax.ShapeDtypeStruct((G, 512), f32),
        scratch_shapes=[
            pltpu.VMEM((G, 512), f32),
            pltpu.VMEM((G, 512), f32),
        ],
    )(z2.reshape(C2, N, DC), dinv16, b2.reshape(1, 1024), Wout,
      bout.reshape(1, 512), batch.reshape(NB, 1, RB).astype(jnp.int32))

    return out
